# baseline (speedup 1.0000x reference)
import jax
import jax.numpy as jnp
from jax.experimental import pallas as pl
from jax.experimental.pallas import tpu as pltpu


def _make_kernel(NB, TB, Q, NW, RB, CH):
    def _fused(x_ref, wr_ref, br_ref, we_ref, be_ref, out_ref,
               acc_ref, xbufs, web0_ref, web1_ref, sb0_ref, sb1_ref,
               xsems, wesems, wsem0, wsem1):
        i = pl.program_id(0)

        def x_copy(blk, slot):
            return pltpu.make_async_copy(
                x_ref.at[pl.ds(blk * TB, TB), :],
                xbufs.at[slot], xsems.at[slot])

        def we_copy(c, webuf):
            return pltpu.make_async_copy(
                we_ref.at[pl.ds(c * CH, CH), :], webuf, wesems.at[c])

        @pl.when(i == 0)
        def _():
            for q in range(Q):
                x_copy(q, q).start()

        @pl.when((i > 0) & (i + Q - 1 < NB))
        def _():
            x_copy(i + Q - 1, (i + Q - 1) % Q).start()

        @pl.when(i < NB)
        def _():
            slot = jax.lax.rem(i, Q)
            x_copy(i, slot).wait()
            xb = xbufs[slot]
            logits = jax.lax.dot_general(
                xb, wr_ref[...], (((1,), (1,)), ((), ())),
                preferred_element_type=jnp.float32)
            logits = logits + br_ref[...]
            m = jnp.max(logits, axis=1, keepdims=True)
            denom = jnp.sum(jnp.exp(logits - m), axis=1, keepdims=True)
            G = 1.0 / denom
            u = jax.lax.dot_general(
                G, xb, (((0,), (0,)), ((), ())),
                preferred_element_type=jnp.float32)
            gsum = jnp.sum(G, axis=0, keepdims=True)
            part = jnp.concatenate(
                [u, jnp.broadcast_to(gsum, u.shape)], axis=0)

            @pl.when(i == 0)
            def _():
                acc_ref[...] = part

            @pl.when(i != 0)
            def _():
                acc_ref[...] += part

        @pl.when(i == NB - 2)
        def _():
            we_copy(0, web0_ref).start()

        def chunk(c, webuf, sbuf, wsem):
            we_copy(c, webuf).wait()
            u = acc_ref[0:1, :]
            g = acc_ref[1, 0]
            v = jax.lax.dot_general(
                u, webuf[...], (((1,), (1,)), ((), ())),
                preferred_element_type=jnp.float32)
            v = v + g * be_ref[0:1, pl.ds(c * CH, CH)]
            sbuf[...] = jnp.broadcast_to(v, sbuf.shape)
            for r in range(NW):
                pltpu.make_async_copy(
                    sbuf,
                    out_ref.at[pl.ds(r * RB, RB), pl.ds(c * CH, CH)],
                    wsem.at[r]).start()

        @pl.when(i == NB)
        def _():
            we_copy(1, web1_ref).start()
            chunk(0, web0_ref, sb0_ref, wsem0)

        @pl.when(i == NB + 1)
        def _():
            chunk(1, web1_ref, sb1_ref, wsem1)
            for r in range(NW):
                pltpu.make_async_copy(
                    sb0_ref,
                    out_ref.at[pl.ds(r * RB, RB), pl.ds(0, CH)],
                    wsem0.at[r]).wait()
                pltpu.make_async_copy(
                    sb1_ref,
                    out_ref.at[pl.ds(r * RB, RB), pl.ds(CH, CH)],
                    wsem1.at[r]).wait()

    return _fused


def kernel(x, W_r, b_r, W_e, b_e):
    b, s, h = x.shape
    bs = b * s
    e = W_r.shape[0]
    xf = x.reshape(bs, h)
    br2 = b_r.reshape(1, e)
    be2 = b_e.reshape(1, h)

    TB = 512
    Q = 6
    RB = 512
    CH = h // 2
    NB = bs // TB
    NW = bs // RB

    yflat = pl.pallas_call(
        _make_kernel(NB, TB, Q, NW, RB, CH),
        grid=(NB + 2,),
        in_specs=[
            pl.BlockSpec(memory_space=pl.ANY),
            pl.BlockSpec((e, h), lambda i: (0, 0)),
            pl.BlockSpec((1, e), lambda i: (0, 0)),
            pl.BlockSpec(memory_space=pl.ANY),
            pl.BlockSpec((1, h), lambda i: (0, 0)),
        ],
        out_specs=pl.BlockSpec(memory_space=pl.ANY),
        out_shape=jax.ShapeDtypeStruct((bs, h), jnp.float32),
        scratch_shapes=[
            pltpu.VMEM((2, h), jnp.float32),
            pltpu.VMEM((Q, TB, h), jnp.float32),
            pltpu.VMEM((CH, h), jnp.float32),
            pltpu.VMEM((CH, h), jnp.float32),
            pltpu.VMEM((RB, CH), jnp.float32),
            pltpu.VMEM((RB, CH), jnp.float32),
            pltpu.SemaphoreType.DMA((Q,)),
            pltpu.SemaphoreType.DMA((2,)),
            pltpu.SemaphoreType.DMA((bs // RB,)),
            pltpu.SemaphoreType.DMA((bs // RB,)),
        ],
        compiler_params=pltpu.CompilerParams(
            dimension_semantics=("arbitrary",)),
    )(xf, W_r, br2, W_e, be2)

    return yflat.reshape(b, s, h)


# speedup vs baseline: 1.0528x; 1.0528x over previous
import jax
import jax.numpy as jnp
from jax.experimental import pallas as pl
from jax.experimental.pallas import tpu as pltpu

_K = 4


def _make_kernel(NR, NW, RB, CH):
    def _fused(x_ref, wr_ref, br_ref, we_ref, be_ref, out_ref,
               acc_ref, webufs, sbufs, wesems, wsems):
        i = pl.program_id(0)

        def we_copy(c):
            return pltpu.make_async_copy(
                we_ref.at[pl.ds(c * CH, CH), :], webufs.at[c], wesems.at[c])

        def stripe_copy(c, r):
            return pltpu.make_async_copy(
                sbufs.at[c],
                out_ref.at[pl.ds(r * RB, RB), pl.ds(c * CH, CH)],
                wsems.at[c, r])

        @pl.when(i < NR)
        def _():
            xb = x_ref[...]
            logits = jax.lax.dot_general(
                xb, wr_ref[...], (((1,), (1,)), ((), ())),
                preferred_element_type=jnp.float32)
            logits = logits + br_ref[...]
            m = jnp.max(logits, axis=1, keepdims=True)
            denom = jnp.sum(jnp.exp(logits - m), axis=1, keepdims=True)
            G = 1.0 / denom
            u = jax.lax.dot_general(
                G, xb, (((0,), (0,)), ((), ())),
                preferred_element_type=jnp.float32)
            gsum = jnp.sum(G, axis=0, keepdims=True)
            part = jnp.concatenate(
                [u, jnp.broadcast_to(gsum, u.shape)], axis=0)

            @pl.when(i == 0)
            def _():
                acc_ref[...] = part

            @pl.when(i != 0)
            def _():
                acc_ref[...] += part

        @pl.when(i == NR - 2)
        def _():
            we_copy(0).start()

        for c in range(_K):
            @pl.when(i == NR + c)
            def _(c=c):
                if c + 1 < _K:
                    we_copy(c + 1).start()
                we_copy(c).wait()
                u = acc_ref[0:1, :]
                g = acc_ref[1, 0]
                v = jax.lax.dot_general(
                    u, webufs[c], (((1,), (1,)), ((), ())),
                    preferred_element_type=jnp.float32)
                v = v + g * be_ref[0:1, pl.ds(c * CH, CH)]
                sbufs.at[c][...] = jnp.broadcast_to(v, (RB, CH))
                for r in range(NW):
                    stripe_copy(c, r).start()
                if c == _K - 1:
                    for cc in range(_K):
                        for r in range(NW):
                            stripe_copy(cc, r).wait()

    return _fused


def kernel(x, W_r, b_r, W_e, b_e):
    b, s, h = x.shape
    bs = b * s
    e = W_r.shape[0]
    xf = x.reshape(bs, h)
    br2 = b_r.reshape(1, e)
    be2 = b_e.reshape(1, h)

    TB = 1024
    RB = 512
    CH = h // _K
    NR = bs // TB
    NW = bs // RB

    yflat = pl.pallas_call(
        _make_kernel(NR, NW, RB, CH),
        grid=(NR + _K,),
        in_specs=[
            pl.BlockSpec((TB, h), lambda i: (jnp.minimum(i, NR - 1), 0)),
            pl.BlockSpec((e, h), lambda i: (0, 0)),
            pl.BlockSpec((1, e), lambda i: (0, 0)),
            pl.BlockSpec(memory_space=pl.ANY),
            pl.BlockSpec((1, h), lambda i: (0, 0)),
        ],
        out_specs=pl.BlockSpec(memory_space=pl.ANY),
        out_shape=jax.ShapeDtypeStruct((bs, h), jnp.float32),
        scratch_shapes=[
            pltpu.VMEM((2, h), jnp.float32),
            pltpu.VMEM((_K, CH, h), jnp.float32),
            pltpu.VMEM((_K, RB, CH), jnp.float32),
            pltpu.SemaphoreType.DMA((_K,)),
            pltpu.SemaphoreType.DMA((_K, bs // RB)),
        ],
        compiler_params=pltpu.CompilerParams(
            dimension_semantics=("arbitrary",)),
    )(xf, W_r, br2, W_e, be2)

    return yflat.reshape(b, s, h)


# speedup vs baseline: 1.0751x; 1.0212x over previous
import jax
import jax.numpy as jnp
from jax.experimental import pallas as pl
from jax.experimental.pallas import tpu as pltpu


def _make_kernel(NR, NW, RB, CH):
    def _fused(x_ref, wr_ref, br_ref, we_ref, be_ref, out_ref,
               acc_ref, web0_ref, web1_ref, sb0_ref, sb1_ref,
               wesems, wsem0, wsem1):
        i = pl.program_id(0)

        def we_copy(c, webuf):
            return pltpu.make_async_copy(
                we_ref.at[pl.ds(c * CH, CH), :], webuf, wesems.at[c])

        @pl.when(i < NR)
        def _():
            xb = x_ref[...]
            logits = jax.lax.dot_general(
                xb, wr_ref[...], (((1,), (1,)), ((), ())),
                preferred_element_type=jnp.float32)
            logits = logits + br_ref[...]
            m = jnp.max(logits, axis=1, keepdims=True)
            denom = jnp.sum(jnp.exp(logits - m), axis=1, keepdims=True)
            G = 1.0 / denom
            u = jax.lax.dot_general(
                G, xb, (((0,), (0,)), ((), ())),
                preferred_element_type=jnp.float32)
            gsum = jnp.sum(G, axis=0, keepdims=True)
            part = jnp.concatenate(
                [u, jnp.broadcast_to(gsum, u.shape)], axis=0)

            @pl.when(i == 0)
            def _():
                acc_ref[...] = part

            @pl.when(i != 0)
            def _():
                acc_ref[...] += part

        @pl.when(i == NR - 2)
        def _():
            we_copy(0, web0_ref).start()

        def chunk(c, webuf, sbuf, wsem):
            we_copy(c, webuf).wait()
            u = acc_ref[0:1, :]
            g = acc_ref[1, 0]
            v = jax.lax.dot_general(
                u, webuf[...], (((1,), (1,)), ((), ())),
                preferred_element_type=jnp.float32)
            v = v + g * be_ref[0:1, pl.ds(c * CH, CH)]
            sbuf[...] = jnp.broadcast_to(v, sbuf.shape)
            for r in range(NW):
                pltpu.make_async_copy(
                    sbuf,
                    out_ref.at[pl.ds(r * RB, RB), pl.ds(c * CH, CH)],
                    wsem.at[r]).start()

        @pl.when(i == NR)
        def _():
            we_copy(1, web1_ref).start()
            chunk(0, web0_ref, sb0_ref, wsem0)

        @pl.when(i == NR + 1)
        def _():
            chunk(1, web1_ref, sb1_ref, wsem1)
            for r in range(NW):
                pltpu.make_async_copy(
                    sb0_ref,
                    out_ref.at[pl.ds(r * RB, RB), pl.ds(0, CH)],
                    wsem0.at[r]).wait()
                pltpu.make_async_copy(
                    sb1_ref,
                    out_ref.at[pl.ds(r * RB, RB), pl.ds(CH, CH)],
                    wsem1.at[r]).wait()

    return _fused


def kernel(x, W_r, b_r, W_e, b_e):
    b, s, h = x.shape
    bs = b * s
    e = W_r.shape[0]
    xf = x.reshape(bs, h)
    br2 = b_r.reshape(1, e)
    be2 = b_e.reshape(1, h)

    TB = 2048
    RB = 512
    CH = h // 2
    NR = bs // TB
    NW = bs // RB

    yflat = pl.pallas_call(
        _make_kernel(NR, NW, RB, CH),
        grid=(NR + 2,),
        in_specs=[
            pl.BlockSpec((TB, h), lambda i: (jnp.minimum(i, NR - 1), 0)),
            pl.BlockSpec((e, h), lambda i: (0, 0)),
            pl.BlockSpec((1, e), lambda i: (0, 0)),
            pl.BlockSpec(memory_space=pl.ANY),
            pl.BlockSpec((1, h), lambda i: (0, 0)),
        ],
        out_specs=pl.BlockSpec(memory_space=pl.ANY),
        out_shape=jax.ShapeDtypeStruct((bs, h), jnp.float32),
        scratch_shapes=[
            pltpu.VMEM((2, h), jnp.float32),
            pltpu.VMEM((CH, h), jnp.float32),
            pltpu.VMEM((CH, h), jnp.float32),
            pltpu.VMEM((RB, CH), jnp.float32),
            pltpu.VMEM((RB, CH), jnp.float32),
            pltpu.SemaphoreType.DMA((2,)),
            pltpu.SemaphoreType.DMA((bs // RB,)),
            pltpu.SemaphoreType.DMA((bs // RB,)),
        ],
        compiler_params=pltpu.CompilerParams(
            dimension_semantics=("arbitrary",)),
    )(xf, W_r, br2, W_e, be2)

    return yflat.reshape(b, s, h)
